# baseline (speedup 1.0000x reference)
import jax
import jax.numpy as jnp
from jax.experimental import pallas as pl
from jax.experimental.pallas import tpu as pltpu

_N = 50000
_D = 256
_M = 512
_B = 1000
_RS = 2
_G = _N // (_B * _RS)


def _mlp_segsum_kernel(x0_ref, x1_ref, w1_ref, b1_ref, w2_ref, b2_ref,
                       w3_ref, b3_ref, ids0_ref, ids1_ref, out_ref,
                       w1s_ref, w2s_ref, w2l_ref):
    i = pl.program_id(0)

    @pl.when(i == 0)
    def _():
        w1s_ref[...] = (w1_ref[...] * 0.5).astype(jnp.bfloat16)
        w2h_f = w2_ref[...] * 0.5
        w2hi = w2h_f.astype(jnp.bfloat16)
        w2s_ref[...] = w2hi
        w2l_ref[...] = (w2h_f - w2hi.astype(jnp.float32)).astype(jnp.bfloat16)
        out_ref[...] = jnp.zeros_like(out_ref)

    b1h = b1_ref[...] * 0.5
    b2h = b2_ref[...] * 0.5
    w3r = w3_ref[...]
    b3 = b3_ref[0, 0]
    w1 = w1s_ref[...]
    w2 = w2s_ref[...]
    w2l = w2l_ref[...]

    partial = jnp.zeros((1, _M), dtype=jnp.float32)
    for blk, (x_ref, ids_ref) in enumerate(((x0_ref, ids0_ref),
                                            (x1_ref, ids1_ref))):
        x = x_ref[...].astype(jnp.bfloat16)
        u = (jnp.dot(x, w1, preferred_element_type=jnp.float32)
             + b1h).astype(jnp.bfloat16)
        t = jnp.tanh(u)
        g = u + u * t
        u = (jnp.dot(g, w2, preferred_element_type=jnp.float32)
             + jnp.dot(g, w2l, preferred_element_type=jnp.float32)
             + b2h).astype(jnp.bfloat16)
        t = jnp.tanh(u)
        h = (u + u * t).astype(jnp.float32)
        s = jnp.sum(h * w3r, axis=1, keepdims=True) + b3

        ids = ids_ref[0, 0, :]
        seg = jax.lax.broadcasted_iota(jnp.int32, (_B, _M), 1)
        hit = ids[:, None] == seg
        partial = partial + jnp.sum(jnp.where(hit, s, 0.0), axis=0,
                                    keepdims=True)

    out_ref[...] += partial


def _x_spec(k):
    return pl.BlockSpec((_B, _D), lambda i, k=k: (_G * k + i, 0))


def _ids_spec(k):
    return pl.BlockSpec((1, 1, _B), lambda i, k=k: (_G * k + i, 0, 0))


def kernel(energy, W1, b1, W2, b2, W3, b3, batch):
    ids3 = batch.astype(jnp.int32).reshape(_N // _B, 1, _B)
    out = pl.pallas_call(
        _mlp_segsum_kernel,
        grid=(_G,),
        in_specs=[
            _x_spec(0),
            _x_spec(1),
            pl.BlockSpec((_D, _D), lambda i: (0, 0)),
            pl.BlockSpec((1, _D), lambda i: (0, 0)),
            pl.BlockSpec((_D, _D), lambda i: (0, 0)),
            pl.BlockSpec((1, _D), lambda i: (0, 0)),
            pl.BlockSpec((1, _D), lambda i: (0, 0)),
            pl.BlockSpec((1, 1), lambda i: (0, 0)),
            _ids_spec(0),
            _ids_spec(1),
        ],
        out_specs=pl.BlockSpec((1, _M), lambda i: (0, 0)),
        out_shape=jax.ShapeDtypeStruct((1, _M), jnp.float32),
        scratch_shapes=[
            pltpu.VMEM((_D, _D), jnp.bfloat16),
            pltpu.VMEM((_D, _D), jnp.bfloat16),
            pltpu.VMEM((_D, _D), jnp.bfloat16),
        ],
    )(energy, energy, W1, b1.reshape(1, _D), W2, b2.reshape(1, _D),
      W3.reshape(1, _D), b3.reshape(1, 1), ids3, ids3)
    return out[0]


# speedup vs baseline: 1.0179x; 1.0179x over previous
import jax
import jax.numpy as jnp
from jax.experimental import pallas as pl
from jax.experimental.pallas import tpu as pltpu

_N = 50000
_D = 256
_M = 512
_B = 1000
_RS = 2
_G = _N // (_B * _RS)


def _mlp_segsum_kernel(x0_ref, x1_ref, w1_ref, b1_ref, w2_ref, b2_ref,
                       w3_ref, b3_ref, ids0_ref, ids1_ref, out_ref,
                       w1s_ref, w2s_ref, w2l_ref):
    i = pl.program_id(0)

    @pl.when(i == 0)
    def _():
        w1s_ref[...] = (w1_ref[...] * 0.5).astype(jnp.bfloat16)
        w2h_f = w2_ref[...] * 0.5
        w2hi = w2h_f.astype(jnp.bfloat16)
        w2s_ref[...] = w2hi
        w2l_ref[...] = (w2h_f - w2hi.astype(jnp.float32)).astype(jnp.bfloat16)
        out_ref[...] = jnp.zeros_like(out_ref)

    b1h = b1_ref[...] * 0.5
    b2h = b2_ref[...] * 0.5
    w3r = w3_ref[...]
    b3 = b3_ref[0, 0]
    w1 = w1s_ref[...]
    w2 = w2s_ref[...]
    w2l = w2l_ref[...]

    partial = jnp.zeros((1, _M), dtype=jnp.float32)
    for blk, (x_ref, ids_ref) in enumerate(((x0_ref, ids0_ref),
                                            (x1_ref, ids1_ref))):
        x = x_ref[...].astype(jnp.bfloat16)
        u = (jnp.dot(x, w1, preferred_element_type=jnp.float32)
             + b1h).astype(jnp.bfloat16)
        t = jnp.tanh(u)
        g = u + u * t
        u = (jnp.dot(g, w2, preferred_element_type=jnp.float32)
             + jnp.dot(g, w2l, preferred_element_type=jnp.float32)
             + b2h)
        t = jnp.tanh(u)
        h = u + u * t
        s = jnp.sum(h * w3r, axis=1, keepdims=True) + b3

        ids = ids_ref[0, 0, :]
        seg = jax.lax.broadcasted_iota(jnp.int32, (_B, _M), 1)
        hit = ids[:, None] == seg
        partial = partial + jnp.sum(jnp.where(hit, s, 0.0), axis=0,
                                    keepdims=True)

    out_ref[...] += partial


def _x_spec(k):
    return pl.BlockSpec((_B, _D), lambda i, k=k: (_G * k + i, 0))


def _ids_spec(k):
    return pl.BlockSpec((1, 1, _B), lambda i, k=k: (_G * k + i, 0, 0))


def kernel(energy, W1, b1, W2, b2, W3, b3, batch):
    ids3 = batch.astype(jnp.int32).reshape(_N // _B, 1, _B)
    out = pl.pallas_call(
        _mlp_segsum_kernel,
        grid=(_G,),
        in_specs=[
            _x_spec(0),
            _x_spec(1),
            pl.BlockSpec((_D, _D), lambda i: (0, 0)),
            pl.BlockSpec((1, _D), lambda i: (0, 0)),
            pl.BlockSpec((_D, _D), lambda i: (0, 0)),
            pl.BlockSpec((1, _D), lambda i: (0, 0)),
            pl.BlockSpec((1, _D), lambda i: (0, 0)),
            pl.BlockSpec((1, 1), lambda i: (0, 0)),
            _ids_spec(0),
            _ids_spec(1),
        ],
        out_specs=pl.BlockSpec((1, _M), lambda i: (0, 0)),
        out_shape=jax.ShapeDtypeStruct((1, _M), jnp.float32),
        scratch_shapes=[
            pltpu.VMEM((_D, _D), jnp.bfloat16),
            pltpu.VMEM((_D, _D), jnp.bfloat16),
            pltpu.VMEM((_D, _D), jnp.bfloat16),
        ],
    )(energy, energy, W1, b1.reshape(1, _D), W2, b2.reshape(1, _D),
      W3.reshape(1, _D), b3.reshape(1, 1), ids3, ids3)
    return out[0]
